# Initial kernel scaffold; baseline (speedup 1.0000x reference)
import functools

import jax
import jax.numpy as jnp
from jax import lax
from jax.experimental import pallas as pl
from jax.experimental.pallas import tpu as pltpu
from jax.experimental.pallas import tpu_sc as plsc

N_NODES = 10000
D_IN = 128
D_OUT = 128
N_EDGES = 320000

NC = 2
NS = 16
NW = NC * NS
CHUNK = 128
DW = 144
CHUNKS_PER_TILE = (N_EDGES + NW * CHUNK - 1) // (NW * CHUNK)
E_PAD = NW * CHUNK * CHUNKS_PER_TILE
ROWS_PER_TILE_ACC = 640
N_ACC = NS * ROWS_PER_TILE_ACC
DUMMY_ROW = N_ACC - 1

ROW_BLOCK = 1000


def _prep_body(x_ref, wl_ref, h_ref):
    h = lax.dot_general(
        x_ref[...], wl_ref[...], (((1,), (1,)), ((), ())),
        preferred_element_type=jnp.float32,
        precision=lax.Precision.HIGHEST,
    )
    h_ref[:, pl.ds(0, D_IN)] = h
    col = lax.broadcasted_iota(jnp.int32, (ROW_BLOCK, DW - D_IN), 1)
    h_ref[:, pl.ds(D_IN, DW - D_IN)] = jnp.where(col == 0, 1.0, 0.0).astype(
        jnp.float32
    )


def _final_body(x_ref, wr_ref, a0_ref, a1_ref, bl_ref, br_ref, out_ref):
    s = a0_ref[...] + a1_ref[...]
    feats = s[:, :D_IN]
    cnt = s[:, D_IN:D_IN + 1]
    agg = feats / jnp.maximum(cnt, 1.0)
    dense = lax.dot_general(
        x_ref[...], wr_ref[...], (((1,), (1,)), ((), ())),
        preferred_element_type=jnp.float32,
        precision=lax.Precision.HIGHEST,
    )
    out_ref[...] = agg + dense + bl_ref[...] + br_ref[...]


def _sc_agg(h, src_idx, dst_idx):
    mesh = plsc.VectorSubcoreMesh(core_axis_name="c", subcore_axis_name="s")

    @functools.partial(
        pl.kernel,
        mesh=mesh,
        out_type=jax.ShapeDtypeStruct((NC, N_ACC, DW), jnp.float32),
        scratch_types=[
            pltpu.VMEM((CHUNKS_PER_TILE, CHUNK), jnp.int32),
            pltpu.VMEM((CHUNKS_PER_TILE, CHUNK), jnp.int32),
            pltpu.VMEM((CHUNK, DW), jnp.float32),
            pltpu.VMEM_SHARED((N_ACC, DW), jnp.float32),
        ],
    )
    def k(h_hbm, src_hbm, dst_hbm, acc_hbm, idx_s, idx_d, gbuf, acc_sh):
        c = lax.axis_index("c")
        s = lax.axis_index("s")
        wid = c * NS + s

        pltpu.sync_copy(src_hbm.at[wid], idx_s)
        pltpu.sync_copy(dst_hbm.at[wid], idx_d)

        @pl.loop(0, CHUNK)
        def _(r):
            for j in range(DW // 16):
                gbuf[r, pl.ds(j * 16, 16)] = jnp.zeros((16,), jnp.float32)

        for kk in range(ROWS_PER_TILE_ACC // CHUNK):
            pltpu.sync_copy(
                gbuf, acc_sh.at[pl.ds(s * ROWS_PER_TILE_ACC + kk * CHUNK, CHUNK)]
            )

        plsc.subcore_barrier()

        @pl.loop(0, CHUNKS_PER_TILE)
        def _(i):
            pltpu.sync_copy(h_hbm.at[idx_s.at[i]], gbuf)
            pltpu.sync_copy(gbuf, acc_sh.at[idx_d.at[i]], add=True)

        plsc.subcore_barrier()

        pltpu.sync_copy(
            acc_sh.at[pl.ds(s * ROWS_PER_TILE_ACC, ROWS_PER_TILE_ACC)],
            acc_hbm.at[c, pl.ds(s * ROWS_PER_TILE_ACC, ROWS_PER_TILE_ACC)],
        )

    return k(h, src_idx, dst_idx)


def kernel(x, edge_index, W_l, b_l, W_r, b_r):
    dst = edge_index[0].astype(jnp.int32)
    src = edge_index[1].astype(jnp.int32)
    n_pad = E_PAD - N_EDGES
    src_p = jnp.concatenate([src, jnp.zeros((n_pad,), jnp.int32)])
    dst_p = jnp.concatenate([dst, jnp.full((n_pad,), DUMMY_ROW, jnp.int32)])
    src_t = src_p.reshape(NW, CHUNKS_PER_TILE, CHUNK)
    dst_t = dst_p.reshape(NW, CHUNKS_PER_TILE, CHUNK)

    grid = N_NODES // ROW_BLOCK
    h = pl.pallas_call(
        _prep_body,
        grid=(grid,),
        in_specs=[
            pl.BlockSpec((ROW_BLOCK, D_IN), lambda i: (i, 0)),
            pl.BlockSpec((D_OUT, D_IN), lambda i: (0, 0)),
        ],
        out_specs=pl.BlockSpec((ROW_BLOCK, DW), lambda i: (i, 0)),
        out_shape=jax.ShapeDtypeStruct((N_NODES, DW), jnp.float32),
    )(x, W_l)

    acc = _sc_agg(h, src_t, dst_t)

    out = pl.pallas_call(
        _final_body,
        grid=(grid,),
        in_specs=[
            pl.BlockSpec((ROW_BLOCK, D_IN), lambda i: (i, 0)),
            pl.BlockSpec((D_OUT, D_IN), lambda i: (0, 0)),
            pl.BlockSpec((ROW_BLOCK, DW), lambda i: (i, 0)),
            pl.BlockSpec((ROW_BLOCK, DW), lambda i: (i, 0)),
            pl.BlockSpec((1, D_OUT), lambda i: (0, 0)),
            pl.BlockSpec((1, D_OUT), lambda i: (0, 0)),
        ],
        out_specs=pl.BlockSpec((ROW_BLOCK, D_OUT), lambda i: (i, 0)),
        out_shape=jax.ShapeDtypeStruct((N_NODES, D_OUT), jnp.float32),
    )(x, W_r, acc[0], acc[1], b_l.reshape(1, D_OUT), b_r.reshape(1, D_OUT))
    return out


# speedup vs baseline: 4.8301x; 4.8301x over previous
import functools

import jax
import jax.numpy as jnp
from jax import lax
from jax.experimental import pallas as pl
from jax.experimental.pallas import tpu as pltpu
from jax.experimental.pallas import tpu_sc as plsc

N_NODES = 10000
D_IN = 128
D_OUT = 128
N_EDGES = 320000

NC = 2
NS = 16
NW = NC * NS
CHUNK = 128
DW = 144
CHUNKS_PER_TILE = (N_EDGES + NW * CHUNK - 1) // (NW * CHUNK)
E_PAD = NW * CHUNK * CHUNKS_PER_TILE
ROWS_PER_TILE_ACC = 640
N_ACC = NS * ROWS_PER_TILE_ACC
DUMMY_ROW = N_ACC - 1

ROW_BLOCK = 1000


def _prep_body(x_ref, wl_ref, h_ref):
    h = lax.dot_general(
        x_ref[...], wl_ref[...], (((1,), (1,)), ((), ())),
        preferred_element_type=jnp.float32,
        precision=lax.Precision.HIGHEST,
    )
    h_ref[:, pl.ds(0, D_IN)] = h
    col = lax.broadcasted_iota(jnp.int32, (ROW_BLOCK, DW - D_IN), 1)
    h_ref[:, pl.ds(D_IN, DW - D_IN)] = jnp.where(col == 0, 1.0, 0.0).astype(
        jnp.float32
    )


def _final_body(x_ref, wr_ref, a0_ref, a1_ref, bl_ref, br_ref, out_ref):
    s = a0_ref[...] + a1_ref[...]
    feats = s[:, :D_IN]
    cnt = s[:, D_IN:D_IN + 1]
    agg = feats / jnp.maximum(cnt, 1.0)
    dense = lax.dot_general(
        x_ref[...], wr_ref[...], (((1,), (1,)), ((), ())),
        preferred_element_type=jnp.float32,
        precision=lax.Precision.HIGHEST,
    )
    out_ref[...] = agg + dense + bl_ref[...] + br_ref[...]


def _sc_agg(h, src_idx, dst_idx):
    mesh = plsc.VectorSubcoreMesh(core_axis_name="c", subcore_axis_name="s")

    @functools.partial(
        pl.kernel,
        mesh=mesh,
        compiler_params=pltpu.CompilerParams(use_tc_tiling_on_sc=False),
        out_type=jax.ShapeDtypeStruct((NC, N_ACC, DW), jnp.float32),
        scratch_types=[
            pltpu.VMEM((CHUNKS_PER_TILE, CHUNK), jnp.int32),
            pltpu.VMEM((CHUNKS_PER_TILE, CHUNK), jnp.int32),
            pltpu.VMEM((CHUNK, DW), jnp.float32),
            pltpu.VMEM_SHARED((N_ACC, DW), jnp.float32),
        ],
    )
    def k(h_hbm, src_hbm, dst_hbm, acc_hbm, idx_s, idx_d, gbuf, acc_sh):
        c = lax.axis_index("c")
        s = lax.axis_index("s")
        wid = c * NS + s

        pltpu.sync_copy(src_hbm.at[wid], idx_s)
        pltpu.sync_copy(dst_hbm.at[wid], idx_d)

        @pl.loop(0, CHUNK)
        def _(r):
            for j in range(DW // 16):
                gbuf[r, pl.ds(j * 16, 16)] = jnp.zeros((16,), jnp.float32)

        for kk in range(ROWS_PER_TILE_ACC // CHUNK):
            pltpu.sync_copy(
                gbuf, acc_sh.at[pl.ds(s * ROWS_PER_TILE_ACC + kk * CHUNK, CHUNK)]
            )

        plsc.subcore_barrier()

        @pl.loop(0, CHUNKS_PER_TILE)
        def _(i):
            pltpu.sync_copy(h_hbm.at[idx_s.at[i]], gbuf)
            pltpu.sync_copy(gbuf, acc_sh.at[idx_d.at[i]], add=True)

        plsc.subcore_barrier()

        pltpu.sync_copy(
            acc_sh.at[pl.ds(s * ROWS_PER_TILE_ACC, ROWS_PER_TILE_ACC)],
            acc_hbm.at[c, pl.ds(s * ROWS_PER_TILE_ACC, ROWS_PER_TILE_ACC)],
        )

    return k(h, src_idx, dst_idx)


def kernel(x, edge_index, W_l, b_l, W_r, b_r):
    dst = edge_index[0].astype(jnp.int32)
    src = edge_index[1].astype(jnp.int32)
    n_pad = E_PAD - N_EDGES
    src_p = jnp.concatenate([src, jnp.zeros((n_pad,), jnp.int32)])
    dst_p = jnp.concatenate([dst, jnp.full((n_pad,), DUMMY_ROW, jnp.int32)])
    src_t = src_p.reshape(NW, CHUNKS_PER_TILE, CHUNK)
    dst_t = dst_p.reshape(NW, CHUNKS_PER_TILE, CHUNK)

    grid = N_NODES // ROW_BLOCK
    h = pl.pallas_call(
        _prep_body,
        grid=(grid,),
        in_specs=[
            pl.BlockSpec((ROW_BLOCK, D_IN), lambda i: (i, 0)),
            pl.BlockSpec((D_OUT, D_IN), lambda i: (0, 0)),
        ],
        out_specs=pl.BlockSpec((ROW_BLOCK, DW), lambda i: (i, 0)),
        out_shape=jax.ShapeDtypeStruct((N_NODES, DW), jnp.float32),
    )(x, W_l)

    acc = _sc_agg(h, src_t, dst_t)

    out = pl.pallas_call(
        _final_body,
        grid=(grid,),
        in_specs=[
            pl.BlockSpec((ROW_BLOCK, D_IN), lambda i: (i, 0)),
            pl.BlockSpec((D_OUT, D_IN), lambda i: (0, 0)),
            pl.BlockSpec((ROW_BLOCK, DW), lambda i: (i, 0)),
            pl.BlockSpec((ROW_BLOCK, DW), lambda i: (i, 0)),
            pl.BlockSpec((1, D_OUT), lambda i: (0, 0)),
            pl.BlockSpec((1, D_OUT), lambda i: (0, 0)),
        ],
        out_specs=pl.BlockSpec((ROW_BLOCK, D_OUT), lambda i: (i, 0)),
        out_shape=jax.ShapeDtypeStruct((N_NODES, D_OUT), jnp.float32),
    )(x, W_r, acc[0], acc[1], b_l.reshape(1, D_OUT), b_r.reshape(1, D_OUT))
    return out
